# baseline (speedup 1.0000x reference)
import functools

import jax
import jax.numpy as jnp
from jax import lax
from jax.experimental import pallas as pl
from jax.experimental.pallas import tpu as pltpu
from jax.experimental.pallas import tpu_sc as plsc

_NUM_CLASSES = 21
_THRESHOLD = 0.5
_NEGPOS_RATIO = 3
_V0, _V1 = 0.1, 0.2

_B = 32
_P = 8732
_O = 10
_SL = 72
_LN = 128
_PP = _SL * _LN
_CHUNKS = 546


_IMGS = 8


def _match_body(pr_ref, tr_ref, loc_ref, ct_ref, lc_ref, misc_ref):
    f32 = jnp.float32
    i32 = jnp.int32
    sub = lax.broadcasted_iota(i32, (_SL, _LN), 0)
    lane = lax.broadcasted_iota(i32, (_SL, _LN), 1)
    flat = sub * _LN + lane
    valid = flat < _P

    cx = pr_ref[0]
    cy = pr_ref[1]
    w = pr_ref[2]
    h = pr_ref[3]
    px0 = cx - w * 0.5
    py0 = cy - h * 0.5
    px1 = cx + w * 0.5
    py1 = cy + h * 0.5
    p_area = w * h

    for img in range(_IMGS):
        ious = []
        for i in range(_O):
            tx0 = tr_ref[img, 0, i]
            ty0 = tr_ref[img, 1, i]
            tx1 = tr_ref[img, 2, i]
            ty1 = tr_ref[img, 3, i]
            t_area = (tx1 - tx0) * (ty1 - ty0)
            ix = jnp.maximum(jnp.minimum(tx1, px1) - jnp.maximum(tx0, px0),
                             0.0)
            iy = jnp.maximum(jnp.minimum(ty1, py1) - jnp.maximum(ty0, py0),
                             0.0)
            inter = ix * iy
            iou = inter / (t_area + p_area - inter)
            ious.append(jnp.where(valid, iou, -1.0))

        bto = ious[0]
        bti = jnp.zeros((_SL, _LN), i32)
        for i in range(1, _O):
            gt = ious[i] > bto
            bti = jnp.where(gt, i, bti)
            bto = jnp.maximum(bto, ious[i])

        maxes = [jnp.max(ious[i]) for i in range(_O)]
        bp = [jnp.min(jnp.where(ious[i] == maxes[i], flat,
                                jnp.int32(2 ** 30)))
              for i in range(_O)]

        for j in range(_O):
            eq = flat == bp[j]
            bto = jnp.where(eq, 2.0, bto)
            bti = jnp.where(eq, j, bti)

        mx0 = jnp.zeros((_SL, _LN), f32)
        my0 = jnp.zeros((_SL, _LN), f32)
        mx1 = jnp.zeros((_SL, _LN), f32)
        my1 = jnp.zeros((_SL, _LN), f32)
        lab = jnp.zeros((_SL, _LN), i32)
        for i in range(_O):
            sel = bti == i
            mx0 = jnp.where(sel, tr_ref[img, 0, i], mx0)
            my0 = jnp.where(sel, tr_ref[img, 1, i], my0)
            mx1 = jnp.where(sel, tr_ref[img, 2, i], mx1)
            my1 = jnp.where(sel, tr_ref[img, 3, i], my1)
            lab = jnp.where(sel, tr_ref[img, 4, i].astype(i32), lab)

        conf = jnp.where(bto < _THRESHOLD, 0, lab + 1)
        conf = jnp.where(valid, conf, 0)
        pos = conf > 0

        g_cx = ((mx0 + mx1) * 0.5 - cx) / (_V0 * w)
        g_cy = ((my0 + my1) * 0.5 - cy) / (_V0 * h)
        g_w = jnp.log((mx1 - mx0) / w) / _V1
        g_h = jnp.log((my1 - my0) / h) / _V1
        acc = jnp.zeros((_SL, _LN), f32)
        for c, g in enumerate((g_cx, g_cy, g_w, g_h)):
            d = loc_ref[img, c] - g
            a = jnp.abs(d)
            acc = acc + jnp.where(a < 1.0, 0.5 * d * d, a - 0.5)
        loss_l = jnp.sum(jnp.where(pos, acc, 0.0))
        num_pos = jnp.sum(jnp.where(pos, 1, 0))

        x0 = ct_ref[img, 0]
        mx = x0
        for c in range(1, _NUM_CLASSES):
            mx = jnp.maximum(mx, ct_ref[img, c])
        s = jnp.exp(x0 - mx)
        for c in range(1, _NUM_CLASSES):
            s = s + jnp.exp(ct_ref[img, c] - mx)
        lse = jnp.log(s) + mx
        pk = jnp.where(conf == 0, x0, 0.0)
        for c in range(1, _NUM_CLASSES):
            pk = jnp.where(conf == c, ct_ref[img, c], pk)
        ce = lse - pk
        sum_pos_ce = jnp.sum(jnp.where(pos, ce, 0.0))

        lc_ref[img] = jnp.where(pos | jnp.logical_not(valid), 0.0,
                                jnp.maximum(ce, 0.0))
        li = lax.broadcasted_iota(i32, (1, _LN), 1)
        misc = jnp.where(li == 0, loss_l,
                         jnp.where(li == 1, num_pos.astype(f32),
                                   jnp.where(li == 2, sum_pos_ce, 0.0)))
        misc_ref[img] = misc


def _match_ce(priors4, tr, locp, ctp):
    return pl.pallas_call(
        _match_body,
        grid=(_B // _IMGS,),
        in_specs=[
            pl.BlockSpec((4, _SL, _LN), lambda b: (0, 0, 0)),
            pl.BlockSpec((_IMGS, 5, 16), lambda b: (b, 0, 0),
                         memory_space=pltpu.SMEM),
            pl.BlockSpec((_IMGS, 4, _SL, _LN), lambda b: (b, 0, 0, 0)),
            pl.BlockSpec((_IMGS, _NUM_CLASSES, _SL, _LN),
                         lambda b: (b, 0, 0, 0)),
        ],
        out_specs=[
            pl.BlockSpec((_IMGS, _SL, _LN), lambda b: (b, 0, 0)),
            pl.BlockSpec((_IMGS, 1, _LN), lambda b: (b, 0, 0)),
        ],
        out_shape=[
            jax.ShapeDtypeStruct((_B, _SL, _LN), jnp.float32),
            jax.ShapeDtypeStruct((_B, 1, _LN), jnp.float32),
        ],
    )(priors4, tr, locp, ctp)


def _mine_body(lc_hbm, misc_hbm, out_hbm, lc_v, misc_v, histc_v, hists_v,
               cand_v, out_v):
    i32 = jnp.int32
    f32 = jnp.float32
    wid = lax.axis_index("s") * 2 + lax.axis_index("c")
    pltpu.sync_copy(lc_hbm.at[wid], lc_v)
    pltpu.sync_copy(misc_hbm.at[wid], misc_v)
    mv = misc_v[pl.ds(0, 16)]
    np_f = mv[1]
    spc = mv[2]
    k = jnp.minimum(_NEGPOS_RATIO * np_f.astype(i32), _P - 1)
    kk = jnp.maximum(k, 1)

    iota16 = lax.broadcasted_iota(i32, (16,), 0)
    ones16 = jnp.full((16,), 1, i32)

    def zero_pass(j, _):
        histc_v[pl.ds(j * 16, 16)] = jnp.zeros((16,), i32)
        hists_v[pl.ds(j * 16, 16)] = jnp.zeros((16,), f32)
        return 0

    lax.fori_loop(0, 128, zero_pass, 0, unroll=8)

    def l0_pass(i, _):
        xf = lc_v[pl.ds(i * 16, 16)]
        xb = lax.bitcast_convert_type(xf, i32)
        d2 = (lax.shift_right_logical(xb, 20) & 0xFF0) | iota16
        plsc.addupdate_scatter(histc_v, [d2], ones16)
        plsc.addupdate_scatter(hists_v, [d2], xf)
        return 0

    lax.fori_loop(0, _CHUNKS, l0_pass, 0, unroll=8)

    def scan0(i, carry):
        acc, found, dstar, cab0, f_above = carry
        d = 127 - i
        c_d = jnp.sum(histc_v[pl.ds(d * 16, 16)])
        f_d = jnp.sum(hists_v[pl.ds(d * 16, 16)])
        pre = found == 0
        cross = pre & ((acc + c_d) >= kk)
        dstar = jnp.where(cross, d, dstar)
        cab0 = jnp.where(cross, acc, cab0)
        f_above = f_above + jnp.where(pre & jnp.logical_not(cross), f_d, 0.0)
        found = found + jnp.where(cross, 1, 0)
        acc = acc + c_d
        return acc, found, dstar, cab0, f_above

    _, _, dstar, cab0, f_above = lax.fori_loop(
        0, 128, scan0,
        (jnp.int32(0), jnp.int32(0), jnp.int32(0), jnp.int32(0),
         jnp.float32(0.0)), unroll=8)
    b0 = dstar
    krem = kk - cab0
    prefix = b0

    def compact_pass(i, off):
        xf = lc_v[pl.ds(i * 16, 16)]
        xb = lax.bitcast_convert_type(xf, i32)
        sel = lax.shift_right_logical(xb, 24) == b0
        plsc.store_compressed(cand_v.at[pl.ds(off, 16)], xf, mask=sel)
        return off + plsc.all_reduce_population_count(sel)[0]

    nc = lax.fori_loop(0, _CHUNKS, compact_pass, jnp.int32(0), unroll=8)
    ncch = (nc + 15) // 16

    for level in range(1, 4):
        sh = 24 - 8 * level
        for j in range(16):
            histc_v[pl.ds(j * 16, 16)] = jnp.zeros((16,), i32)

        def hist_pass(i, _, sh=sh, prefix=prefix):
            xb = lax.bitcast_convert_type(cand_v[pl.ds(i * 16, 16)], i32)
            intail = i * 16 + iota16 < nc
            match = (lax.shift_right_logical(xb, sh + 8) == prefix) & intail
            d = lax.shift_right_logical(xb, sh) & 255
            plsc.addupdate_scatter(histc_v, [d], ones16, mask=match)
            return 0

        lax.fori_loop(0, ncch, hist_pass, 0)

        acc = jnp.int32(0)
        found = jnp.int32(0)
        dstar = jnp.int32(0)
        cab = jnp.int32(0)
        for j in reversed(range(16)):
            hch = histc_v[pl.ds(j * 16, 16)]
            rev = jnp.flip(hch, axis=0)
            cs = jnp.cumsum(rev)
            m = ((acc + cs) >= krem) & ((acc + cs - rev) < krem) & (found == 0)
            anyc = jnp.sum(jnp.where(m, 1, 0))
            dstar = dstar + jnp.sum(jnp.where(m, 16 * j + 15 - iota16, 0))
            cab = cab + jnp.sum(jnp.where(m, acc + cs - rev, 0))
            found = found + anyc
            acc = acc + jnp.sum(hch)
        prefix = jnp.where(found > 0, (prefix << 8) | dstar, prefix)
        krem = jnp.where(found > 0, krem - cab, krem)

    tbits = prefix
    tval = lax.bitcast_convert_type(tbits, f32)

    def sum_pass(i, carry):
        s, c = carry
        xf = cand_v[pl.ds(i * 16, 16)]
        xb = lax.bitcast_convert_type(xf, i32)
        gt = (xb > tbits) & (i * 16 + iota16 < nc)
        s = s + jnp.sum(jnp.where(gt, xf, 0.0))
        c = c + jnp.sum(jnp.where(gt, 1, 0))
        return s, c

    s_c, c_c = lax.fori_loop(0, ncch, sum_pass,
                             (jnp.float32(0.0), jnp.int32(0)))
    krem1 = kk - cab0
    topk = f_above + s_c + (krem1 - c_c).astype(f32) * tval
    topk = jnp.where(k > 0, topk, 0.0)
    row_conf = spc + topk

    o = jnp.where(iota16 == 0, row_conf, 0.0)
    out_v[...] = o
    pltpu.sync_copy(out_v, out_hbm.at[wid])


def _mine(lc2, misc2):
    mesh = plsc.VectorSubcoreMesh(core_axis_name="c", subcore_axis_name="s")
    f = functools.partial(
        pl.kernel,
        out_type=jax.ShapeDtypeStruct((_B, 16), jnp.float32),
        mesh=mesh,
        scratch_types=[
            pltpu.VMEM((_PP,), jnp.float32),
            pltpu.VMEM((_LN,), jnp.float32),
            pltpu.VMEM((4096,), jnp.int32),
            pltpu.VMEM((4096,), jnp.float32),
            pltpu.VMEM((_PP,), jnp.float32),
            pltpu.VMEM((16,), jnp.float32),
        ],
        compiler_params=pltpu.CompilerParams(needs_layout_passes=False),
    )(_mine_body)
    return f(lc2, misc2)


def _combine_body(misc_ref, sc_ref, out_ref):
    i32 = jnp.int32
    mi = misc_ref[...]
    li = lax.broadcasted_iota(i32, (_B, 1, _LN), 2)
    ll = jnp.sum(jnp.where(li == 0, mi, 0.0))
    npt = jnp.sum(jnp.where(li == 1, mi, 0.0))
    sc = sc_ref[...]
    li2 = lax.broadcasted_iota(i32, (_B, 16), 1)
    lc = jnp.sum(jnp.where(li2 == 0, sc, 0.0))
    n = jnp.maximum(npt, 1.0)
    lo = lax.broadcasted_iota(i32, (1, _LN), 1)
    out_ref[...] = jnp.where(lo == 0, ll / n, jnp.where(lo == 1, lc / n, 0.0))


def _combine(misc, sc_out):
    return pl.pallas_call(
        _combine_body,
        in_specs=[
            pl.BlockSpec((_B, 1, _LN), lambda: (0, 0, 0)),
            pl.BlockSpec((_B, 16), lambda: (0, 0)),
        ],
        out_specs=pl.BlockSpec((1, _LN), lambda: (0, 0)),
        out_shape=jax.ShapeDtypeStruct((1, _LN), jnp.float32),
    )(misc, sc_out)


def kernel(loc_data, conf_data, priors, targets):
    f32 = jnp.float32
    pad_pr = jnp.broadcast_to(jnp.array([0.0, 0.0, 1.0, 1.0], f32),
                              (_PP - _P, 4))
    priors4 = jnp.concatenate([priors, pad_pr], axis=0).T.reshape(4, _SL, _LN)
    tr = jnp.pad(jnp.transpose(targets, (0, 2, 1)), ((0, 0), (0, 0), (0, 6)))
    locp = jnp.pad(jnp.transpose(loc_data, (0, 2, 1)),
                   ((0, 0), (0, 0), (0, _PP - _P))).reshape(_B, 4, _SL, _LN)

    ctp = jnp.pad(jnp.transpose(conf_data, (0, 2, 1)),
                  ((0, 0), (0, 0), (0, _PP - _P))).reshape(
                      _B, _NUM_CLASSES, _SL, _LN)

    lc, misc = _match_ce(priors4, tr, locp, ctp)
    sc_out = _mine(lc.reshape(_B, _PP), misc.reshape(_B, _LN))
    out = _combine(misc, sc_out)
    return out[0, 0], out[0, 1]


# speedup vs baseline: 1.0033x; 1.0033x over previous
import functools

import jax
import jax.numpy as jnp
from jax import lax
from jax.experimental import pallas as pl
from jax.experimental.pallas import tpu as pltpu
from jax.experimental.pallas import tpu_sc as plsc

_NUM_CLASSES = 21
_THRESHOLD = 0.5
_NEGPOS_RATIO = 3
_V0, _V1 = 0.1, 0.2

_B = 32
_P = 8732
_O = 10
_SL = 72
_LN = 128
_PP = _SL * _LN
_CHUNKS = 546


_IMGS = 4


def _match_body(pr_ref, tr_ref, loc_ref, ct_ref, lc_ref, misc_ref):
    f32 = jnp.float32
    i32 = jnp.int32
    sub = lax.broadcasted_iota(i32, (_SL, _LN), 0)
    lane = lax.broadcasted_iota(i32, (_SL, _LN), 1)
    flat = sub * _LN + lane
    valid = flat < _P

    cx = pr_ref[0]
    cy = pr_ref[1]
    w = pr_ref[2]
    h = pr_ref[3]
    px0 = cx - w * 0.5
    py0 = cy - h * 0.5
    px1 = cx + w * 0.5
    py1 = cy + h * 0.5
    p_area = w * h

    for img in range(_IMGS):
        ious = []
        for i in range(_O):
            tx0 = tr_ref[img, 0, i]
            ty0 = tr_ref[img, 1, i]
            tx1 = tr_ref[img, 2, i]
            ty1 = tr_ref[img, 3, i]
            t_area = (tx1 - tx0) * (ty1 - ty0)
            ix = jnp.maximum(jnp.minimum(tx1, px1) - jnp.maximum(tx0, px0),
                             0.0)
            iy = jnp.maximum(jnp.minimum(ty1, py1) - jnp.maximum(ty0, py0),
                             0.0)
            inter = ix * iy
            iou = inter / (t_area + p_area - inter)
            ious.append(jnp.where(valid, iou, -1.0))

        bto = ious[0]
        bti = jnp.zeros((_SL, _LN), i32)
        for i in range(1, _O):
            gt = ious[i] > bto
            bti = jnp.where(gt, i, bti)
            bto = jnp.maximum(bto, ious[i])

        maxes = [jnp.max(ious[i]) for i in range(_O)]
        bp = [jnp.min(jnp.where(ious[i] == maxes[i], flat,
                                jnp.int32(2 ** 30)))
              for i in range(_O)]

        for j in range(_O):
            eq = flat == bp[j]
            bto = jnp.where(eq, 2.0, bto)
            bti = jnp.where(eq, j, bti)

        mx0 = jnp.zeros((_SL, _LN), f32)
        my0 = jnp.zeros((_SL, _LN), f32)
        mx1 = jnp.zeros((_SL, _LN), f32)
        my1 = jnp.zeros((_SL, _LN), f32)
        lab = jnp.zeros((_SL, _LN), i32)
        for i in range(_O):
            sel = bti == i
            mx0 = jnp.where(sel, tr_ref[img, 0, i], mx0)
            my0 = jnp.where(sel, tr_ref[img, 1, i], my0)
            mx1 = jnp.where(sel, tr_ref[img, 2, i], mx1)
            my1 = jnp.where(sel, tr_ref[img, 3, i], my1)
            lab = jnp.where(sel, tr_ref[img, 4, i].astype(i32), lab)

        conf = jnp.where(bto < _THRESHOLD, 0, lab + 1)
        conf = jnp.where(valid, conf, 0)
        pos = conf > 0

        g_cx = ((mx0 + mx1) * 0.5 - cx) / (_V0 * w)
        g_cy = ((my0 + my1) * 0.5 - cy) / (_V0 * h)
        g_w = jnp.log((mx1 - mx0) / w) / _V1
        g_h = jnp.log((my1 - my0) / h) / _V1
        acc = jnp.zeros((_SL, _LN), f32)
        for c, g in enumerate((g_cx, g_cy, g_w, g_h)):
            d = loc_ref[img, c] - g
            a = jnp.abs(d)
            acc = acc + jnp.where(a < 1.0, 0.5 * d * d, a - 0.5)
        loss_l = jnp.sum(jnp.where(pos, acc, 0.0))
        num_pos = jnp.sum(jnp.where(pos, 1, 0))

        x0 = ct_ref[img, 0]
        mx = x0
        for c in range(1, _NUM_CLASSES):
            mx = jnp.maximum(mx, ct_ref[img, c])
        s = jnp.exp(x0 - mx)
        for c in range(1, _NUM_CLASSES):
            s = s + jnp.exp(ct_ref[img, c] - mx)
        lse = jnp.log(s) + mx
        pk = jnp.where(conf == 0, x0, 0.0)
        for c in range(1, _NUM_CLASSES):
            pk = jnp.where(conf == c, ct_ref[img, c], pk)
        ce = lse - pk
        sum_pos_ce = jnp.sum(jnp.where(pos, ce, 0.0))

        lc_ref[img] = jnp.where(pos | jnp.logical_not(valid), 0.0,
                                jnp.maximum(ce, 0.0))
        li = lax.broadcasted_iota(i32, (1, _LN), 1)
        misc = jnp.where(li == 0, loss_l,
                         jnp.where(li == 1, num_pos.astype(f32),
                                   jnp.where(li == 2, sum_pos_ce, 0.0)))
        misc_ref[img] = misc


def _match_ce(priors4, tr, locp, ctp):
    return pl.pallas_call(
        _match_body,
        grid=(_B // _IMGS,),
        in_specs=[
            pl.BlockSpec((4, _SL, _LN), lambda b: (0, 0, 0)),
            pl.BlockSpec((_IMGS, 5, 16), lambda b: (b, 0, 0),
                         memory_space=pltpu.SMEM),
            pl.BlockSpec((_IMGS, 4, _SL, _LN), lambda b: (b, 0, 0, 0)),
            pl.BlockSpec((_IMGS, _NUM_CLASSES, _SL, _LN),
                         lambda b: (b, 0, 0, 0)),
        ],
        out_specs=[
            pl.BlockSpec((_IMGS, _SL, _LN), lambda b: (b, 0, 0)),
            pl.BlockSpec((_IMGS, 1, _LN), lambda b: (b, 0, 0)),
        ],
        out_shape=[
            jax.ShapeDtypeStruct((_B, _SL, _LN), jnp.float32),
            jax.ShapeDtypeStruct((_B, 1, _LN), jnp.float32),
        ],
    )(priors4, tr, locp, ctp)


def _mine_body(lc_hbm, misc_hbm, out_hbm, lc_v, misc_v, histc_v, hists_v,
               cand_v, out_v):
    i32 = jnp.int32
    f32 = jnp.float32
    wid = lax.axis_index("s") * 2 + lax.axis_index("c")
    pltpu.sync_copy(lc_hbm.at[wid], lc_v)
    pltpu.sync_copy(misc_hbm.at[wid], misc_v)
    mv = misc_v[pl.ds(0, 16)]
    np_f = mv[1]
    spc = mv[2]
    k = jnp.minimum(_NEGPOS_RATIO * np_f.astype(i32), _P - 1)
    kk = jnp.maximum(k, 1)

    iota16 = lax.broadcasted_iota(i32, (16,), 0)
    ones16 = jnp.full((16,), 1, i32)

    def zero_pass(j, _):
        histc_v[pl.ds(j * 16, 16)] = jnp.zeros((16,), i32)
        hists_v[pl.ds(j * 16, 16)] = jnp.zeros((16,), f32)
        return 0

    lax.fori_loop(0, 128, zero_pass, 0, unroll=8)

    def l0_pass(i, _):
        xf = lc_v[pl.ds(i * 16, 16)]
        xb = lax.bitcast_convert_type(xf, i32)
        d2 = (lax.shift_right_logical(xb, 20) & 0xFF0) | iota16
        plsc.addupdate_scatter(histc_v, [d2], ones16)
        plsc.addupdate_scatter(hists_v, [d2], xf)
        return 0

    lax.fori_loop(0, _CHUNKS, l0_pass, 0, unroll=8)

    def scan0(i, carry):
        acc, found, dstar, cab0, f_above = carry
        d = 127 - i
        c_d = jnp.sum(histc_v[pl.ds(d * 16, 16)])
        f_d = jnp.sum(hists_v[pl.ds(d * 16, 16)])
        pre = found == 0
        cross = pre & ((acc + c_d) >= kk)
        dstar = jnp.where(cross, d, dstar)
        cab0 = jnp.where(cross, acc, cab0)
        f_above = f_above + jnp.where(pre & jnp.logical_not(cross), f_d, 0.0)
        found = found + jnp.where(cross, 1, 0)
        acc = acc + c_d
        return acc, found, dstar, cab0, f_above

    _, _, dstar, cab0, f_above = lax.fori_loop(
        0, 128, scan0,
        (jnp.int32(0), jnp.int32(0), jnp.int32(0), jnp.int32(0),
         jnp.float32(0.0)), unroll=8)
    b0 = dstar
    krem = kk - cab0
    prefix = b0

    def compact_pass(i, off):
        xf = lc_v[pl.ds(i * 16, 16)]
        xb = lax.bitcast_convert_type(xf, i32)
        sel = lax.shift_right_logical(xb, 24) == b0
        plsc.store_compressed(cand_v.at[pl.ds(off, 16)], xf, mask=sel)
        return off + plsc.all_reduce_population_count(sel)[0]

    nc = lax.fori_loop(0, _CHUNKS, compact_pass, jnp.int32(0), unroll=8)
    ncch = (nc + 15) // 16

    for level in range(1, 4):
        sh = 24 - 8 * level
        for j in range(16):
            histc_v[pl.ds(j * 16, 16)] = jnp.zeros((16,), i32)

        def hist_pass(i, _, sh=sh, prefix=prefix):
            xb = lax.bitcast_convert_type(cand_v[pl.ds(i * 16, 16)], i32)
            intail = i * 16 + iota16 < nc
            match = (lax.shift_right_logical(xb, sh + 8) == prefix) & intail
            d = lax.shift_right_logical(xb, sh) & 255
            plsc.addupdate_scatter(histc_v, [d], ones16, mask=match)
            return 0

        lax.fori_loop(0, ncch, hist_pass, 0)

        acc = jnp.int32(0)
        found = jnp.int32(0)
        dstar = jnp.int32(0)
        cab = jnp.int32(0)
        for j in reversed(range(16)):
            hch = histc_v[pl.ds(j * 16, 16)]
            rev = jnp.flip(hch, axis=0)
            cs = jnp.cumsum(rev)
            m = ((acc + cs) >= krem) & ((acc + cs - rev) < krem) & (found == 0)
            anyc = jnp.sum(jnp.where(m, 1, 0))
            dstar = dstar + jnp.sum(jnp.where(m, 16 * j + 15 - iota16, 0))
            cab = cab + jnp.sum(jnp.where(m, acc + cs - rev, 0))
            found = found + anyc
            acc = acc + jnp.sum(hch)
        prefix = jnp.where(found > 0, (prefix << 8) | dstar, prefix)
        krem = jnp.where(found > 0, krem - cab, krem)

    tbits = prefix
    tval = lax.bitcast_convert_type(tbits, f32)

    def sum_pass(i, carry):
        s, c = carry
        xf = cand_v[pl.ds(i * 16, 16)]
        xb = lax.bitcast_convert_type(xf, i32)
        gt = (xb > tbits) & (i * 16 + iota16 < nc)
        s = s + jnp.sum(jnp.where(gt, xf, 0.0))
        c = c + jnp.sum(jnp.where(gt, 1, 0))
        return s, c

    s_c, c_c = lax.fori_loop(0, ncch, sum_pass,
                             (jnp.float32(0.0), jnp.int32(0)))
    krem1 = kk - cab0
    topk = f_above + s_c + (krem1 - c_c).astype(f32) * tval
    topk = jnp.where(k > 0, topk, 0.0)
    row_conf = spc + topk

    o = jnp.where(iota16 == 0, row_conf, 0.0)
    out_v[...] = o
    pltpu.sync_copy(out_v, out_hbm.at[wid])


def _mine(lc2, misc2):
    mesh = plsc.VectorSubcoreMesh(core_axis_name="c", subcore_axis_name="s")
    f = functools.partial(
        pl.kernel,
        out_type=jax.ShapeDtypeStruct((_B, 16), jnp.float32),
        mesh=mesh,
        scratch_types=[
            pltpu.VMEM((_PP,), jnp.float32),
            pltpu.VMEM((_LN,), jnp.float32),
            pltpu.VMEM((4096,), jnp.int32),
            pltpu.VMEM((4096,), jnp.float32),
            pltpu.VMEM((_PP,), jnp.float32),
            pltpu.VMEM((16,), jnp.float32),
        ],
        compiler_params=pltpu.CompilerParams(needs_layout_passes=False),
    )(_mine_body)
    return f(lc2, misc2)


def _combine_body(misc_ref, sc_ref, out_ref):
    i32 = jnp.int32
    mi = misc_ref[...]
    li = lax.broadcasted_iota(i32, (_B, 1, _LN), 2)
    ll = jnp.sum(jnp.where(li == 0, mi, 0.0))
    npt = jnp.sum(jnp.where(li == 1, mi, 0.0))
    sc = sc_ref[...]
    li2 = lax.broadcasted_iota(i32, (_B, 16), 1)
    lc = jnp.sum(jnp.where(li2 == 0, sc, 0.0))
    n = jnp.maximum(npt, 1.0)
    lo = lax.broadcasted_iota(i32, (1, _LN), 1)
    out_ref[...] = jnp.where(lo == 0, ll / n, jnp.where(lo == 1, lc / n, 0.0))


def _combine(misc, sc_out):
    return pl.pallas_call(
        _combine_body,
        in_specs=[
            pl.BlockSpec((_B, 1, _LN), lambda: (0, 0, 0)),
            pl.BlockSpec((_B, 16), lambda: (0, 0)),
        ],
        out_specs=pl.BlockSpec((1, _LN), lambda: (0, 0)),
        out_shape=jax.ShapeDtypeStruct((1, _LN), jnp.float32),
    )(misc, sc_out)


def kernel(loc_data, conf_data, priors, targets):
    f32 = jnp.float32
    pad_pr = jnp.broadcast_to(jnp.array([0.0, 0.0, 1.0, 1.0], f32),
                              (_PP - _P, 4))
    priors4 = jnp.concatenate([priors, pad_pr], axis=0).T.reshape(4, _SL, _LN)
    tr = jnp.pad(jnp.transpose(targets, (0, 2, 1)), ((0, 0), (0, 0), (0, 6)))
    locp = jnp.pad(jnp.transpose(loc_data, (0, 2, 1)),
                   ((0, 0), (0, 0), (0, _PP - _P))).reshape(_B, 4, _SL, _LN)

    ctp = jnp.pad(jnp.transpose(conf_data, (0, 2, 1)),
                  ((0, 0), (0, 0), (0, _PP - _P))).reshape(
                      _B, _NUM_CLASSES, _SL, _LN)

    lc, misc = _match_ce(priors4, tr, locp, ctp)
    sc_out = _mine(lc.reshape(_B, _PP), misc.reshape(_B, _LN))
    out = _combine(misc, sc_out)
    return out[0, 0], out[0, 1]
